# baseline (speedup 1.0000x reference)
import jax
import jax.numpy as jnp
from jax.experimental import pallas as pl
from jax.experimental.pallas import tpu as pltpu

_N_NODES = 32
_G = 8
_F32 = jnp.float32


def _dot(a, b):
    return jax.lax.dot_general(a, b, (((1,), (0,)), ((), ())),
                               preferred_element_type=_F32)


def _body(*refs):
    x_ref, ei_ref, ew_ref = refs[0], refs[1], refs[2]
    o_ref = refs[-1]
    wrefs = refs[3:-1]
    nb = (len(wrefs) - 4) // 6
    n = _N_NODES
    G = _G
    E = ei_ref.shape[1]

    rows = jax.lax.broadcasted_iota(jnp.int32, (n, E), 0)
    S = jnp.where(ei_ref[0:1, :] == rows, 1.0, 0.0).astype(_F32)
    Dw = jnp.where(ei_ref[1:2, :] == rows, ew_ref[0:1, :], 0.0).astype(_F32)
    At = jax.lax.dot_general(S, Dw, (((1,), (1,)), ((), ())),
                             preferred_element_type=_F32)

    M0 = x_ref.shape[0]
    cin0 = x_ref.shape[3]
    cur = x_ref[...].reshape(M0, G * n, cin0)

    def tconv(cur, Wr, br):
        k, cin, c3 = Wr.shape
        c = c3 // 3
        M = cur.shape[0]
        Mo = M - k + 1
        acc = None
        for t in range(k):
            sl = cur[t:t + Mo].reshape(Mo * G * n, cin)
            pp = _dot(sl, Wr[t])
            acc = pp if acc is None else acc + pp
        acc = acc + br[...]
        P = acc[:, :c]
        Q = acc[:, c:2 * c]
        R = acc[:, 2 * c:]
        out = jnp.maximum(P * jax.nn.sigmoid(Q) + R, 0.0)
        return out.reshape(Mo, G * n, c)

    for b in range(nb):
        t1W, t1b, gW, gb, t2W, t2b = wrefs[6 * b:6 * b + 6]
        cur = tconv(cur, t1W, t1b)
        M, _, h = cur.shape
        z = jnp.swapaxes(cur.reshape(M, G, n, h), 2, 3)
        z = _dot(z.reshape(M * G * h, n), At).reshape(M, G, h, n)
        z = jnp.swapaxes(z, 2, 3).reshape(M * G * n, h)
        z = jnp.maximum(_dot(z, gW[...]) + gb[...], 0.0)
        cur = tconv(z.reshape(M, G * n, h), t2W, t2b)

    cw, cb, fw, fb = wrefs[-4:]
    y = _dot(cur[0], cw[0]) + _dot(cur[1], cw[1]) + cb[...]
    pooled = jnp.mean(y.reshape(G, n, -1), axis=1)
    r = jnp.maximum(pooled, 0.0)
    o_ref[...] = (_dot(r, fw[...]) + fb[...]).reshape(G, 1, 1)


def kernel(x, edge_index, edge_attr, batch, params):
    n = _N_NODES
    ng = x.shape[0] // n
    E = edge_index.shape[1]

    cin0 = params["blocks"][0]["t1"]["w1"].shape[2]
    m0 = x.shape[1] // cin0
    xt = x.reshape(ng, n, m0, cin0).transpose(2, 0, 1, 3)

    ins = [xt, edge_index.astype(jnp.int32), edge_attr.reshape(1, E)]
    for blk in params["blocks"]:
        for tk in ("t1", "t2"):
            tp = blk[tk]
            W = jnp.concatenate(
                [tp["w1"][:, 0], tp["w2"][:, 0], tp["w3"][:, 0]], axis=-1)
            bcat = jnp.concatenate([tp["b1"], tp["b2"], tp["b3"]])
            if tk == "t1":
                ins += [W, bcat.reshape(1, -1),
                        blk["gW"], blk["gb"].reshape(1, -1)]
            else:
                ins += [W, bcat.reshape(1, -1)]
    ins += [params["conv_w"], params["conv_b"].reshape(1, -1),
            params["fc_w"], params["fc_b"].reshape(1, 1)]

    def const_spec(a):
        return pl.BlockSpec(a.shape, lambda g: (0,) * a.ndim)

    in_specs = [pl.BlockSpec((m0, _G, n, cin0), lambda g: (0, g, 0, 0))]
    in_specs += [const_spec(a) for a in ins[1:]]

    out = pl.pallas_call(
        _body,
        grid=(ng // _G,),
        in_specs=in_specs,
        out_specs=pl.BlockSpec((_G, 1, 1), lambda g: (g, 0, 0)),
        out_shape=jax.ShapeDtypeStruct((ng, 1, 1), _F32),
        compiler_params=pltpu.CompilerParams(
            dimension_semantics=("parallel",)),
    )(*ins)
    return out.reshape(ng, 1)


# speedup vs baseline: 1.1666x; 1.1666x over previous
import jax
import jax.numpy as jnp
from jax.experimental import pallas as pl
from jax.experimental.pallas import tpu as pltpu

_N_NODES = 32
_G = 8
_F32 = jnp.float32


def _dot(a, b):
    return jax.lax.dot_general(a, b, (((1,), (0,)), ((), ())),
                               preferred_element_type=_F32)


def _dot16(a, b):
    return jax.lax.dot_general(a.astype(jnp.bfloat16), b.astype(jnp.bfloat16),
                               (((1,), (0,)), ((), ())),
                               preferred_element_type=_F32)


def _body(*refs):
    x_ref, ei_ref, ew_ref = refs[0], refs[1], refs[2]
    o_ref = refs[-1]
    wrefs = refs[3:-1]
    nb = (len(wrefs) - 4) // 6
    n = _N_NODES
    G = _G
    E = ei_ref.shape[1]

    rows = jax.lax.broadcasted_iota(jnp.int32, (n, E), 0)
    S = jnp.where(ei_ref[0:1, :] == rows, 1.0, 0.0).astype(_F32)
    Dw = jnp.where(ei_ref[1:2, :] == rows, ew_ref[0:1, :], 0.0).astype(_F32)
    At = jax.lax.dot_general(S, Dw, (((1,), (1,)), ((), ())),
                             preferred_element_type=_F32)

    cin0 = wrefs[0].shape[1]
    M0 = x_ref.shape[1] // cin0
    cur = x_ref[...].reshape(G * n, M0, cin0)
    cur = jnp.transpose(cur, (1, 0, 2))

    def tconv(cur, Wr, br):
        k, cin, c3 = Wr.shape
        c = c3 // 3
        M = cur.shape[0]
        Mo = M - k + 1
        acc = None
        for t in range(k):
            sl = cur[t:t + Mo].reshape(Mo * G * n, cin)
            pp = _dot16(sl, Wr[t])
            acc = pp if acc is None else acc + pp
        acc = acc + br[...]
        P = acc[:, :c]
        Q = acc[:, c:2 * c]
        R = acc[:, 2 * c:]
        out = jnp.maximum(P * jax.nn.sigmoid(Q) + R, 0.0)
        return out.reshape(Mo, G * n, c)

    for b in range(nb):
        t1W, t1b, gW, gb, t2W, t2b = wrefs[6 * b:6 * b + 6]
        cur = tconv(cur, t1W, t1b)
        M, _, h = cur.shape
        z = jnp.swapaxes(cur.reshape(M, G, n, h), 2, 3)
        z = _dot16(z.reshape(M * G * h, n), At).reshape(M, G, h, n)
        z = jnp.swapaxes(z, 2, 3).reshape(M * G * n, h)
        z = jnp.maximum(_dot16(z, gW[...]) + gb[...], 0.0)
        cur = tconv(z.reshape(M, G * n, h), t2W, t2b)

    cw, cb, fw, fb = wrefs[-4:]
    y = _dot16(cur[0], cw[0]) + _dot16(cur[1], cw[1]) + cb[...]
    pooled = jnp.mean(y.reshape(G, n, -1), axis=1)
    r = jnp.maximum(pooled, 0.0)
    o_ref[...] = (_dot(r, fw[...]) + fb[...]).reshape(G, 1, 1)


def kernel(x, edge_index, edge_attr, batch, params):
    n = _N_NODES
    ng = x.shape[0] // n
    E = edge_index.shape[1]

    ins = [x, edge_index.astype(jnp.int32), edge_attr.reshape(1, E)]
    for blk in params["blocks"]:
        for tk in ("t1", "t2"):
            tp = blk[tk]
            W = jnp.concatenate(
                [tp["w1"][:, 0], tp["w2"][:, 0], tp["w3"][:, 0]],
                axis=-1).astype(jnp.bfloat16)
            bcat = jnp.concatenate([tp["b1"], tp["b2"], tp["b3"]])
            if tk == "t1":
                ins += [W, bcat.reshape(1, -1),
                        blk["gW"].astype(jnp.bfloat16),
                        blk["gb"].reshape(1, -1)]
            else:
                ins += [W, bcat.reshape(1, -1)]
    ins += [params["conv_w"].astype(jnp.bfloat16),
            params["conv_b"].reshape(1, -1),
            params["fc_w"], params["fc_b"].reshape(1, 1)]

    def const_spec(a):
        return pl.BlockSpec(a.shape, lambda g: (0,) * a.ndim)

    in_specs = [pl.BlockSpec((_G * n, x.shape[1]), lambda g: (g, 0))]
    in_specs += [const_spec(a) for a in ins[1:]]

    out = pl.pallas_call(
        _body,
        grid=(ng // _G,),
        in_specs=in_specs,
        out_specs=pl.BlockSpec((_G, 1, 1), lambda g: (g, 0, 0)),
        out_shape=jax.ShapeDtypeStruct((ng, 1, 1), _F32),
        compiler_params=pltpu.CompilerParams(
            dimension_semantics=("parallel",)),
    )(*ins)
    return out.reshape(ng, 1)
